# baseline (speedup 1.0000x reference)
import functools

import jax
import jax.numpy as jnp
import numpy as np
from jax import lax
from jax.experimental import pallas as pl
from jax.experimental.pallas import tpu as pltpu

_IMG = 16
_VMEM = 50 * 1024 * 1024


def _enc_kernel(x_ref, t1_ref, b1_ref, t2_ref, b2_ref, o_ref):
    IMG = x_ref.shape[0]
    C = x_ref.shape[1]
    H = x_ref.shape[2]
    W = x_ref.shape[3]
    H4 = H // 4
    Mq = IMG * H4

    zrow = jnp.zeros((IMG, 1, W), jnp.bfloat16)

    def piece(c, j, d):
        start = d + j - 1
        if start < 0:
            body = x_ref[:, c, pl.ds(3, H4 - 1, 4), :].astype(jnp.bfloat16)
            return jnp.concatenate([zrow, body], axis=1)
        if start + 4 * (H4 - 1) >= H:
            body = x_ref[:, c, pl.ds(start, H4 - 1, 4), :].astype(jnp.bfloat16)
            return jnp.concatenate([body, zrow], axis=1)
        return x_ref[:, c, pl.ds(start, H4, 4), :].astype(jnp.bfloat16)

    t1 = t1_ref[...]
    b1 = b1_ref[...]
    n1 = b1.shape[1] // 2

    lhs4 = jnp.concatenate(
        [jnp.concatenate([piece(c, j, d) for d in range(3) for c in range(C)],
                         axis=2).reshape(Mq, 3 * C * W)
         for j in range(4)], axis=0)
    y = jnp.maximum(
        jnp.dot(lhs4, t1, preferred_element_type=jnp.float32) + b1, 0.0)
    yw = jnp.maximum(y[:, :n1], y[:, n1:])
    pe = jnp.maximum(yw[0:Mq], yw[Mq:2 * Mq]).astype(jnp.bfloat16)
    po = jnp.maximum(yw[2 * Mq:3 * Mq], yw[3 * Mq:4 * Mq]).astype(jnp.bfloat16)
    pe3 = pe.reshape(IMG, H4, n1)
    po3 = po.reshape(IMG, H4, n1)

    zrow1 = jnp.zeros((IMG, 1, n1), jnp.bfloat16)
    po_dn = jnp.concatenate([zrow1, po3[:, :H4 - 1, :]], axis=1)
    pe_up = jnp.concatenate([pe3[:, 1:, :], zrow1], axis=1)

    t2 = t2_ref[...]
    b2 = b2_ref[...]
    n2 = b2.shape[1] // 2

    lhs2 = jnp.concatenate(
        [jnp.concatenate(p, axis=2).reshape(Mq, 3 * n1)
         for p in ([po_dn, pe3, po3], [pe3, po3, pe_up])], axis=0)
    y2 = jnp.maximum(
        jnp.dot(lhs2, t2, preferred_element_type=jnp.float32) + b2, 0.0)
    y2w = jnp.maximum(y2[:, :n2], y2[:, n2:])
    out = jnp.maximum(y2w[0:Mq], y2w[Mq:2 * Mq])
    o_ref[...] = out.reshape(IMG, H4, n2).astype(jnp.bfloat16)


def _head_kernel(f1_ref, f2_ref, f3_ref, o_ref, *, ns, nq, way, inv_temp):
    x = jnp.concatenate([f1_ref[...], f2_ref[...]], axis=0)
    p16 = f3_ref[...]
    ns_p = f1_ref.shape[0]
    xf = x.astype(jnp.float32)
    sq = jnp.sum(xf * xf, axis=1, keepdims=True)
    pf = p16.astype(jnp.float32)
    sqp = jnp.transpose(jnp.sum(pf * pf, axis=1, keepdims=True))

    dn = (((1,), (1,)), ((), ()))
    cross = lax.dot_general(x, p16, dn, preferred_element_type=jnp.float32)
    dist = jnp.sqrt(jnp.maximum(sq + sqp - 2.0 * cross, 0.0))

    mins = jnp.minimum(dist[:, 0:way], dist[:, way:2 * way])
    sg = mins[:ns, :]
    qg = mins[ns_p:ns_p + nq, :]
    ab = lax.dot_general(qg, sg, dn, preferred_element_type=jnp.float32)
    sqa = jnp.sum(qg * qg, axis=1, keepdims=True)
    sqb = jnp.transpose(jnp.sum(sg * sg, axis=1, keepdims=True))
    o_ref[...] = -(sqa + sqb - 2.0 * ab) * inv_temp


def _tbuild_kernel(ts1_ref, wo1_ref, ts2_ref, wo2_ref, t1_ref, t2_ref):
    def build(ts_ref, wo_ref, t_ref, cin, w_in):
        kb = t_ref.shape[1]
        wo = wo_ref[0:1, :]
        p = lax.broadcasted_iota(jnp.int32, (w_in, kb), 0)
        for y in range(3):
            for c in range(cin):
                acc = jnp.zeros((w_in, kb), jnp.float32)
                for d in range(3):
                    row = ts_ref[(y * 3 + d) * cin + c:(y * 3 + d) * cin + c + 1, :]
                    acc = jnp.where(p == wo + (d - 1), row.astype(jnp.float32), acc)
                r0 = (y * cin + c) * w_in
                t_ref[r0:r0 + w_in, :] = acc.astype(t_ref.dtype)

    build(ts1_ref, wo1_ref, t1_ref, ts1_ref.shape[0] // 9, t1_ref.shape[0] * 3 // ts1_ref.shape[0])
    build(ts2_ref, wo2_ref, t2_ref, ts2_ref.shape[0] // 9, t2_ref.shape[0] * 3 // ts2_ref.shape[0])


def _col_maps(w_out, cout):
    half = w_out // 2
    wo, co = [], []
    for h in (0, 1):
        for c in range(cout):
            for q in range(half):
                wo.append(2 * q + h)
                co.append(c)
    return np.asarray(wo), np.asarray(co)


def kernel(data_shot, data_query, protos,
           conv1_w_ee, conv1_w_oe, conv1_w_eo, conv1_w_oo, conv1_bias,
           conv2_w_ee, conv2_w_oe, conv2_w_eo, conv2_w_oo, conv2_bias):
    ns, C, H, W = data_shot.shape
    nq = data_query.shape[0]
    npro = protos.shape[0]
    way, temperature = 5, 16.0
    hid = conv2_w_ee.shape[2] // (W // 4)

    taps1 = jnp.stack([conv1_w_ee[:, 0:C, 0:hid],
                       conv1_w_oe[:, 0:C, 0:hid],
                       conv1_w_ee[:, C:2 * C, 0:hid]], axis=1)
    taps2 = jnp.stack([conv2_w_ee[:, 0:hid, 0:hid],
                       conv2_w_oe[:, 0:hid, 0:hid],
                       conv2_w_ee[:, hid:2 * hid, 0:hid]], axis=1)
    wo1_np, co1 = _col_maps(W, hid)
    wo2_np, co2 = _col_maps(W // 2, hid)
    K1, K2 = W * hid, (W // 2) * hid

    def expand(taps, cin, w_out):
        half = jnp.broadcast_to(taps[..., None],
                                taps.shape + (w_out // 2,)).reshape(
                                    3, 3, cin, hid * (w_out // 2))
        return jnp.concatenate([half, half], axis=-1).reshape(
            9 * cin, hid * w_out)

    ts1 = expand(taps1, C, W)
    ts2 = expand(taps2, hid, W // 2)
    wo1 = jnp.asarray(np.tile(wo1_np[None, :], (8, 1)), jnp.int32)
    wo2 = jnp.asarray(np.tile(wo2_np[None, :], (8, 1)), jnp.int32)
    t1, t2 = pl.pallas_call(
        _tbuild_kernel,
        out_shape=[jax.ShapeDtypeStruct((3 * C * W, K1), jnp.bfloat16),
                   jax.ShapeDtypeStruct((3 * hid * (W // 2), K2), jnp.bfloat16)],
        grid=(2,),
        in_specs=[
            pl.BlockSpec((9 * C, K1 // 2), lambda i: (0, i)),
            pl.BlockSpec((8, K1 // 2), lambda i: (0, i)),
            pl.BlockSpec((9 * hid, K2 // 2), lambda i: (0, i)),
            pl.BlockSpec((8, K2 // 2), lambda i: (0, i)),
        ],
        out_specs=[pl.BlockSpec((3 * C * W, K1 // 2), lambda i: (0, i)),
                   pl.BlockSpec((3 * hid * (W // 2), K2 // 2), lambda i: (0, i))],
        compiler_params=pltpu.CompilerParams(
            dimension_semantics=("parallel",),
            vmem_limit_bytes=_VMEM),
    )(ts1, wo1, ts2, wo2)
    def expand_b(bias, w_out):
        half = jnp.broadcast_to(bias[0, 0:hid, None],
                                (hid, w_out // 2)).reshape(1, hid * (w_out // 2))
        return jnp.concatenate([half, half], axis=-1)

    b1 = expand_b(conv1_bias, W)
    b2 = expand_b(conv2_bias, W // 2)

    H4, W4 = H // 4, W // 4
    Nc2 = W4 * hid

    def encode(x):
        n = x.shape[0]
        blocks = pl.cdiv(n, _IMG)
        return pl.pallas_call(
            _enc_kernel,
            out_shape=jax.ShapeDtypeStruct((blocks * _IMG, H4, Nc2), jnp.bfloat16),
            grid=(blocks,),
            in_specs=[
                pl.BlockSpec((_IMG, C, H, W), lambda n: (n, 0, 0, 0)),
                pl.BlockSpec(t1.shape, lambda n: (0, 0)),
                pl.BlockSpec(b1.shape, lambda n: (0, 0)),
                pl.BlockSpec(t2.shape, lambda n: (0, 0)),
                pl.BlockSpec(b2.shape, lambda n: (0, 0)),
            ],
            out_specs=pl.BlockSpec((_IMG, H4, Nc2), lambda n: (n, 0, 0)),
            compiler_params=pltpu.CompilerParams(
                dimension_semantics=("parallel",),
                vmem_limit_bytes=_VMEM),
        )(x, t1, b1, t2, b2)

    D = H4 * Nc2
    f1 = encode(data_shot).reshape(-1, D)
    f2 = encode(data_query).reshape(-1, D)
    f3 = encode(protos).reshape(-1, D)

    head = functools.partial(_head_kernel, ns=ns, nq=nq, way=way,
                             inv_temp=float(1.0 / temperature))
    logits = pl.pallas_call(
        head,
        out_shape=jax.ShapeDtypeStruct((nq, ns), jnp.float32),
        grid=(1,),
        in_specs=[
            pl.BlockSpec(f1.shape, lambda i: (0, 0)),
            pl.BlockSpec(f2.shape, lambda i: (0, 0)),
            pl.BlockSpec(f3.shape, lambda i: (0, 0)),
        ],
        out_specs=pl.BlockSpec((nq, ns), lambda i: (0, 0)),
        compiler_params=pltpu.CompilerParams(
            dimension_semantics=("arbitrary",),
            vmem_limit_bytes=_VMEM),
    )(f1, f2, f3)
    return logits
